# baseline (speedup 1.0000x reference)
from functools import partial

import jax
import jax.numpy as jnp
from jax.experimental import pallas as pl
from jax.experimental.pallas import tpu as pltpu


def _ghost_fused_kernel(x_ref, pww_ref, pwb_ref, cw_ref, cb_ref,
                        dww_ref, dwb_ref, o_ref,
                        x1p_ref, x2p_ref, sp_ref,
                        *, H, W, L, Ho, Wo):
    xs = x_ref[...].reshape(x_ref.shape[0], H * W)
    wv = pww_ref[...]

    x1 = jax.lax.dot_general(xs, wv, (((0,), (0,)), ((), ())),
                             preferred_element_type=jnp.float32)
    x1 = jnp.maximum(x1 + pwb_ref[...], 0.0)
    x1 = x1.reshape(H, W, L)

    zrow = jnp.zeros((1, W + 2, L), jnp.float32)
    zcol = jnp.zeros((H + 2, 1, L), jnp.float32)
    for ref in (x1p_ref, x2p_ref):
        ref[0:1] = zrow
        ref[H + 1:H + 2] = zrow
        ref[:, 0:1] = zcol
        ref[:, W + 1:W + 2] = zcol

    x1p_ref[1:H + 1, 1:W + 1, :] = x1

    cwv = cw_ref[...]
    cbv = cb_ref[...].reshape(1, 1, L)
    TB = 8
    for t in range(H // TB):
        r0 = TB * t
        acc = jnp.zeros((TB, W, L), jnp.float32)
        for ky in range(3):
            for kx in range(3):
                acc = acc + (x1p_ref[r0 + ky:r0 + ky + TB, kx:kx + W, :]
                             * cwv[ky, kx].reshape(1, 1, L))
        x2p_ref[r0 + 1:r0 + 1 + TB, 1:W + 1, :] = jnp.maximum(acc + cbv, 0.0)

    dwv = dww_ref[...]
    HB = Ho // 2
    for half, src in ((0, x1p_ref), (1, x2p_ref)):
        for b in range(2):
            h0 = HB * b
            sacc = jnp.zeros((HB, Wo, L), jnp.float32)
            for ky in range(3):
                for kx in range(3):
                    taps = src[pl.ds(2 * h0 + ky, HB, stride=2),
                               pl.ds(kx, Wo, stride=2), :]
                    sacc = sacc + taps * dwv[ky, kx, half].reshape(1, 1, L)
            sp_ref[h0:h0 + HB, 0:Wo, :] = sacc + dwb_ref[half].reshape(1, 1, L)
        v = sp_ref[...].reshape(Ho * 128, L)
        t = jnp.transpose(v)
        t3 = t.reshape(L, Ho, 128)
        o_ref[0, :, half] = t3[:, :, 0:Wo].reshape(2, L // 2, Ho, Wo)


def kernel(x_nchw, pw_w, pw_scale, pw_bias, cheap_w, cheap_scale, cheap_bias,
           dw_w, dw_scale, dw_bias):
    N, Cin, H, W = x_nchw.shape
    C = pw_w.shape[1]
    L = 2 * C
    N2 = N // 2
    Ho = (H - 1) // 2 + 1
    Wo = (W - 1) // 2 + 1

    pw_eff = pw_w * pw_scale.reshape(1, C)
    cw_eff = cheap_w * cheap_scale.reshape(1, 1, C)
    dw_eff = dw_w * dw_scale.reshape(1, 1, 2 * C)

    z = jnp.zeros((Cin, C), jnp.float32)
    w2 = jnp.concatenate([jnp.concatenate([pw_eff, z], axis=1),
                          jnp.concatenate([z, pw_eff], axis=1)], axis=0)
    tile2 = lambda v: jnp.tile(v.reshape(1, -1), (1, 2))

    body = partial(_ghost_fused_kernel, H=H, W=W, L=L, Ho=Ho, Wo=Wo)
    out6 = pl.pallas_call(
        body,
        out_shape=jax.ShapeDtypeStruct((N2, 2, 2, C, Ho, Wo), jnp.float32),
        grid=(N2,),
        in_specs=[
            pl.BlockSpec((2 * Cin, H, W), lambda n: (n, 0, 0)),
            pl.BlockSpec((2 * Cin, L), lambda n: (0, 0)),
            pl.BlockSpec((1, L), lambda n: (0, 0)),
            pl.BlockSpec((3, 3, L), lambda n: (0, 0, 0)),
            pl.BlockSpec((1, L), lambda n: (0, 0)),
            pl.BlockSpec((3, 3, 2, L), lambda n: (0, 0, 0, 0)),
            pl.BlockSpec((2, L), lambda n: (0, 0)),
        ],
        out_specs=pl.BlockSpec((1, 2, 2, C, Ho, Wo),
                               lambda n: (n, 0, 0, 0, 0, 0)),
        scratch_shapes=[
            pltpu.VMEM((H + 2, W + 2, L), jnp.float32),
            pltpu.VMEM((H + 2, W + 2, L), jnp.float32),
            pltpu.VMEM((Ho, 128, L), jnp.float32),
        ],
        compiler_params=pltpu.CompilerParams(
            dimension_semantics=("parallel",),
            vmem_limit_bytes=48 * 1024 * 1024),
    )(x_nchw.reshape(N2 * 2 * Cin, H, W), w2,
      tile2(pw_bias),
      jnp.tile(cw_eff, (1, 1, 2)), tile2(cheap_bias),
      jnp.tile(dw_eff.reshape(3, 3, 2, C), (1, 1, 1, 2)),
      jnp.tile(dw_bias.reshape(2, C), (1, 2)))

    return out6.reshape(N, 2 * C, Ho, Wo)


# speedup vs baseline: 2.3357x; 2.3357x over previous
from functools import partial

import jax
import jax.numpy as jnp
from jax.experimental import pallas as pl
from jax.experimental.pallas import tpu as pltpu


def _ghost_fused_kernel(x_ref, wl_ref, wr_ref, pwb_ref, cw_ref, cb_ref,
                        dww_ref, dwb_ref, o_ref,
                        x1p_ref, x2p_ref, *, H, W, L, Ho, Wo):
    C = L // 2
    xs0 = x_ref[0].reshape(H * W, C)
    xs1 = x_ref[1].reshape(H * W, C)

    x1 = (jnp.dot(xs0, wl_ref[...], preferred_element_type=jnp.float32)
          + jnp.dot(xs1, wr_ref[...], preferred_element_type=jnp.float32))
    x1 = jnp.maximum(x1 + pwb_ref[...], 0.0)
    x1 = x1.reshape(H, W, L)

    zrow = jnp.zeros((1, W + 2, L), jnp.float32)
    zcol = jnp.zeros((H + 2, 1, L), jnp.float32)
    for ref in (x1p_ref, x2p_ref):
        ref[0:1] = zrow
        ref[H + 1:H + 2] = zrow
        ref[:, 0:1] = zcol
        ref[:, W + 1:W + 2] = zcol

    x1p_ref[1:H + 1, 1:W + 1, :] = x1

    cwv = cw_ref[...]
    cbv = cb_ref[...].reshape(1, 1, L)
    TB = 8
    for t in range(H // TB):
        r0 = TB * t
        acc = jnp.zeros((TB, W, L), jnp.float32)
        for ky in range(3):
            for kx in range(3):
                acc = acc + (x1p_ref[r0 + ky:r0 + ky + TB, kx:kx + W, :]
                             * cwv[ky, kx].reshape(1, 1, L))
        x2p_ref[r0 + 1:r0 + 1 + TB, 1:W + 1, :] = jnp.maximum(acc + cbv, 0.0)

    dwv = dww_ref[...]
    halves = []
    for half, src in ((0, x1p_ref), (1, x2p_ref)):
        sacc = jnp.zeros((Ho, Wo, L), jnp.float32)
        for ky in range(3):
            for kx in range(3):
                taps = src[pl.ds(ky, Ho, stride=2),
                           pl.ds(kx, Wo, stride=2), :]
                sacc = sacc + taps * dwv[ky, kx, half].reshape(1, 1, L)
        halves.append(sacc + dwb_ref[half].reshape(1, 1, L))

    s0, s1 = halves
    per_img = [jnp.concatenate([s0[:, :, C * i:C * i + C],
                                s1[:, :, C * i:C * i + C]], axis=-1)
               for i in (0, 1)]
    o_ref[:, :, 0] = jnp.stack(per_img, axis=2)


def kernel(x_nchw, pw_w, pw_scale, pw_bias, cheap_w, cheap_scale, cheap_bias,
           dw_w, dw_scale, dw_bias):
    N, Cin, H, W = x_nchw.shape
    C = pw_w.shape[1]
    L = 2 * C
    N2 = N // 2
    Ho = (H - 1) // 2 + 1
    Wo = (W - 1) // 2 + 1

    x_nhwc = jnp.transpose(x_nchw, (0, 2, 3, 1))

    pw_eff = pw_w * pw_scale.reshape(1, C)
    cw_eff = cheap_w * cheap_scale.reshape(1, 1, C)
    dw_eff = dw_w * dw_scale.reshape(1, 1, 2 * C)

    z = jnp.zeros((Cin, C), jnp.float32)
    wl = jnp.concatenate([pw_eff, z], axis=1)
    wr = jnp.concatenate([z, pw_eff], axis=1)
    tile2 = lambda v: jnp.tile(v.reshape(1, -1), (1, 2))

    body = partial(_ghost_fused_kernel, H=H, W=W, L=L, Ho=Ho, Wo=Wo)
    out4 = pl.pallas_call(
        body,
        out_shape=jax.ShapeDtypeStruct((Ho, Wo, N2, 2, L), jnp.float32),
        grid=(N2,),
        in_specs=[
            pl.BlockSpec((2, H, W, Cin), lambda n: (n, 0, 0, 0)),
            pl.BlockSpec((Cin, L), lambda n: (0, 0)),
            pl.BlockSpec((Cin, L), lambda n: (0, 0)),
            pl.BlockSpec((1, L), lambda n: (0, 0)),
            pl.BlockSpec((3, 3, L), lambda n: (0, 0, 0)),
            pl.BlockSpec((1, L), lambda n: (0, 0)),
            pl.BlockSpec((3, 3, 2, L), lambda n: (0, 0, 0, 0)),
            pl.BlockSpec((2, L), lambda n: (0, 0)),
        ],
        out_specs=pl.BlockSpec((Ho, Wo, 1, 2, L), lambda n: (0, 0, n, 0, 0)),
        scratch_shapes=[
            pltpu.VMEM((H + 2, W + 2, L), jnp.float32),
            pltpu.VMEM((H + 2, W + 2, L), jnp.float32),
        ],
        compiler_params=pltpu.CompilerParams(
            dimension_semantics=("parallel",),
            vmem_limit_bytes=64 * 1024 * 1024),
    )(x_nhwc, wl, wr,
      tile2(pw_bias),
      jnp.tile(cw_eff, (1, 1, 2)), tile2(cheap_bias),
      jnp.tile(dw_eff.reshape(3, 3, 2, C), (1, 1, 1, 2)),
      jnp.tile(dw_bias.reshape(2, C), (1, 2)))

    return jnp.transpose(out4.reshape(Ho, Wo, N, L), (2, 3, 0, 1))


# speedup vs baseline: 2.7363x; 1.1715x over previous
from functools import partial

import jax
import jax.numpy as jnp
from jax.experimental import pallas as pl
from jax.experimental.pallas import tpu as pltpu


def _ghost_fused_kernel(x_ref, pww_ref, pws_ref, pwb_ref, cw_ref, cs_ref,
                        cb_ref, dww_ref, dws_ref, dwb_ref, o_ref,
                        x1p_ref, x2p_ref, *, H, W, L, Ho, Wo):
    C = L // 2
    xs0 = x_ref[0].reshape(H * W, C)
    xs1 = x_ref[1].reshape(H * W, C)

    pw_eff = pww_ref[...] * pws_ref[...]
    z = jnp.zeros_like(pw_eff)
    wl = jnp.concatenate([pw_eff, z], axis=1)
    wr = jnp.concatenate([z, pw_eff], axis=1)
    pwb = jnp.concatenate([pwb_ref[...], pwb_ref[...]], axis=1)

    x1 = (jnp.dot(xs0, wl, preferred_element_type=jnp.float32)
          + jnp.dot(xs1, wr, preferred_element_type=jnp.float32))
    x1 = jnp.maximum(x1 + pwb, 0.0)
    x1 = x1.reshape(H, W, L)

    zrow = jnp.zeros((1, W + 2, L), jnp.float32)
    zcol = jnp.zeros((H + 2, 1, L), jnp.float32)
    for ref in (x1p_ref, x2p_ref):
        ref[0:1] = zrow
        ref[H + 1:H + 2] = zrow
        ref[:, 0:1] = zcol
        ref[:, W + 1:W + 2] = zcol

    x1p_ref[1:H + 1, 1:W + 1, :] = x1

    cw1 = cw_ref[...] * cs_ref[...].reshape(1, 1, C)
    cwv = jnp.concatenate([cw1, cw1], axis=2)
    cb1 = cb_ref[...]
    cbv = jnp.concatenate([cb1, cb1], axis=1).reshape(1, 1, L)
    TB = 8
    for t in range(H // TB):
        r0 = TB * t
        acc = jnp.zeros((TB, W, L), jnp.float32)
        for ky in range(3):
            for kx in range(3):
                acc = acc + (x1p_ref[r0 + ky:r0 + ky + TB, kx:kx + W, :]
                             * cwv[ky, kx].reshape(1, 1, L))
        x2p_ref[r0 + 1:r0 + 1 + TB, 1:W + 1, :] = jnp.maximum(acc + cbv, 0.0)

    dwe = dww_ref[...] * dws_ref[...].reshape(1, 1, L)
    dwb = dwb_ref[...]
    halves = []
    for half, src in ((0, x1p_ref), (1, x2p_ref)):
        dh = dwe[:, :, C * half:C * half + C]
        dwv = jnp.concatenate([dh, dh], axis=2)
        bh = dwb[:, C * half:C * half + C]
        bhv = jnp.concatenate([bh, bh], axis=1).reshape(1, 1, L)
        sacc = jnp.zeros((Ho, Wo, L), jnp.float32)
        for ky in range(3):
            for kx in range(3):
                taps = src[pl.ds(ky, Ho, stride=2),
                           pl.ds(kx, Wo, stride=2), :]
                sacc = sacc + taps * dwv[ky, kx].reshape(1, 1, L)
        halves.append(sacc + bhv)

    s0, s1 = halves
    per_img = [jnp.concatenate([s0[:, :, C * i:C * i + C],
                                s1[:, :, C * i:C * i + C]], axis=-1)
               for i in (0, 1)]
    o_ref[:, :, 0] = jnp.stack(per_img, axis=2)


def kernel(x_nchw, pw_w, pw_scale, pw_bias, cheap_w, cheap_scale, cheap_bias,
           dw_w, dw_scale, dw_bias):
    N, Cin, H, W = x_nchw.shape
    C = pw_w.shape[1]
    L = 2 * C
    N2 = N // 2
    Ho = (H - 1) // 2 + 1
    Wo = (W - 1) // 2 + 1

    x_nhwc = jnp.transpose(x_nchw, (0, 2, 3, 1))

    body = partial(_ghost_fused_kernel, H=H, W=W, L=L, Ho=Ho, Wo=Wo)
    out4 = pl.pallas_call(
        body,
        out_shape=jax.ShapeDtypeStruct((Ho, Wo, N2, 2, L), jnp.float32),
        grid=(N2,),
        in_specs=[
            pl.BlockSpec((2, H, W, Cin), lambda n: (n, 0, 0, 0)),
            pl.BlockSpec((Cin, C), lambda n: (0, 0)),
            pl.BlockSpec((1, C), lambda n: (0, 0)),
            pl.BlockSpec((1, C), lambda n: (0, 0)),
            pl.BlockSpec((3, 3, C), lambda n: (0, 0, 0)),
            pl.BlockSpec((1, C), lambda n: (0, 0)),
            pl.BlockSpec((1, C), lambda n: (0, 0)),
            pl.BlockSpec((3, 3, L), lambda n: (0, 0, 0)),
            pl.BlockSpec((1, L), lambda n: (0, 0)),
            pl.BlockSpec((1, L), lambda n: (0, 0)),
        ],
        out_specs=pl.BlockSpec((Ho, Wo, 1, 2, L), lambda n: (0, 0, n, 0, 0)),
        scratch_shapes=[
            pltpu.VMEM((H + 2, W + 2, L), jnp.float32),
            pltpu.VMEM((H + 2, W + 2, L), jnp.float32),
        ],
        compiler_params=pltpu.CompilerParams(
            dimension_semantics=("parallel",),
            vmem_limit_bytes=64 * 1024 * 1024),
    )(x_nhwc, pw_w, pw_scale.reshape(1, C), pw_bias.reshape(1, C),
      cheap_w, cheap_scale.reshape(1, C), cheap_bias.reshape(1, C),
      dw_w, dw_scale.reshape(1, L), dw_bias.reshape(1, L))

    return jnp.transpose(out4.reshape(Ho, Wo, N, L), (2, 3, 0, 1))
